# baseline (speedup 1.0000x reference)
import jax
import jax.numpy as jnp
from jax import lax
from jax.experimental import pallas as pl
from jax.experimental.pallas import tpu as pltpu
from jax.experimental.pallas import tpu_sc as plsc

B = 16384
H = 50
LEN_RULES = 100000
PAD_TOK = LEN_RULES
TBL_PAD = 100032
TBL_W = TBL_PAD // 2
TBL_Q = TBL_W // 4
NC, NS, L = 2, 16, 16
NW = NC * NS
ROWS_PER_W = B // NW
CHUNK = 128
NCHUNK = ROWS_PER_W // CHUNK
GROUPS_PER_CHUNK = CHUNK // L
NACC = 4


def _body(rules_hbm, table_hbm, out_hbm, table_v, idx0, idx1, out_v,
          sem_t, sem0, sem1):
    wid = lax.axis_index("s") * NC + lax.axis_index("c")
    base = wid * ROWS_PER_W
    lanes = lax.iota(jnp.int32, L)

    idx_bufs = [idx0, idx1]
    sems = [sem0, sem1]

    def start(c):
        return pltpu.async_copy(
            rules_hbm.at[pl.ds(base + c * CHUNK, CHUNK)],
            idx_bufs[c % 2], sems[c % 2])

    with jax.named_scope("stage_start"):
        cp_t = [
            pltpu.async_copy(table_hbm.at[pl.ds(q * TBL_Q, TBL_Q)],
                             table_v.at[pl.ds(q * TBL_Q, TBL_Q)], sem_t)
            for q in range(4)
        ]
        cps = {0: start(0), 1: start(1)}
        for cp in cp_t:
            cp.wait()

    for c in range(NCHUNK):
        with jax.named_scope("chunk"):
            cps[c].wait()
            if c + 2 < NCHUNK:
                cps[c + 2] = start(c + 2)
            idx_v = idx_bufs[c % 2]

            @plsc.parallel_loop(0, GROUPS_PER_CHUNK, 1)
            def group(g):
                rows = g * L + lanes
                acc = [jnp.ones((L,), jnp.float32) for _ in range(NACC)]
                for j in range(H):
                    iv = plsc.load_gather(
                        idx_v, [rows, jnp.full((L,), j, jnp.int32)])
                    w = plsc.load_gather(table_v, [iv >> 1])
                    hi = w & jnp.int32(-65536)
                    lo = w << 16
                    bits = jnp.where((iv & 1) == 1, hi, lo)
                    v = plsc.bitcast(bits, jnp.float32)
                    acc[j % NACC] = acc[j % NACC] * v
                p = (acc[0] * acc[1]) * (acc[2] * acc[3])
                no = 1.0 - 1.0 / p
                no = jnp.minimum(jnp.maximum(no, 0.0001), 0.99999)
                out_v[pl.ds(c * CHUNK + g * L, L)] = no

    with jax.named_scope("store_out"):
        pltpu.sync_copy(out_v, out_hbm.at[pl.ds(base, ROWS_PER_W)])


@jax.jit
def kernel(rules, relation, table):
    del relation
    tbl = 1.0 + jnp.exp(table[:, 0].at[PAD_TOK].set(-jnp.inf))
    tbl = jnp.concatenate(
        [tbl, jnp.ones((TBL_PAD - (LEN_RULES + 1),), jnp.float32)])
    packed = jax.lax.bitcast_convert_type(
        tbl.astype(jnp.bfloat16).reshape(TBL_W, 2), jnp.int32)
    run = pl.kernel(
        _body,
        out_type=jax.ShapeDtypeStruct((B,), jnp.float32),
        mesh=plsc.VectorSubcoreMesh(
            core_axis_name="c", subcore_axis_name="s",
            num_cores=NC, num_subcores=NS,
        ),
        compiler_params=pltpu.CompilerParams(
            needs_layout_passes=False, use_tc_tiling_on_sc=True),
        scratch_types=[
            pltpu.VMEM((TBL_W,), jnp.int32),
            pltpu.VMEM((CHUNK, H), jnp.int32),
            pltpu.VMEM((CHUNK, H), jnp.int32),
            pltpu.VMEM((ROWS_PER_W,), jnp.float32),
            pltpu.SemaphoreType.DMA,
            pltpu.SemaphoreType.DMA,
            pltpu.SemaphoreType.DMA,
        ],
    )
    return run(rules, packed).reshape(B, 1)


# speedup vs baseline: 2.1459x; 2.1459x over previous
import jax
import jax.numpy as jnp
from jax import lax
from jax.experimental import pallas as pl
from jax.experimental.pallas import tpu as pltpu
from jax.experimental.pallas import tpu_sc as plsc

B = 16384
H = 50
LEN_RULES = 100000
PAD_TOK = LEN_RULES
TBL_PAD = 100096
TBL_S = TBL_PAD // 16
NC, NS, L = 2, 16, 16
NW = NC * NS
ROWS_PER_W = B // NW
CHUNK = 64
NCHUNK = ROWS_PER_W // CHUNK
GROUPS_PER_CHUNK = CHUNK // L
NACC = 4


def _body(rules_hbm, table_hbm, out_hbm, table_v, idx0, idx1, out_v, tbl_sh,
          sem_t, sem0, sem1):
    sid = lax.axis_index("s")
    wid = sid * NC + lax.axis_index("c")
    base = wid * ROWS_PER_W
    lanes = lax.iota(jnp.int32, L)

    idx_bufs = [idx0, idx1]
    sems = [sem0, sem1]

    def start(c):
        return pltpu.async_copy(
            rules_hbm.at[pl.ds(base + c * CHUNK, CHUNK), pl.ds(0, H)],
            idx_bufs[c % 2], sems[c % 2])

    with jax.named_scope("stage_spmem"):
        cp_t = pltpu.async_copy(table_hbm.at[pl.ds(sid * TBL_S, TBL_S)],
                                table_v.at[pl.ds(0, TBL_S)], sem_t)
        cps = {0: start(0), 1: start(1)}
        cp_t.wait()
        pltpu.sync_copy(table_v.at[pl.ds(0, TBL_S)],
                        tbl_sh.at[pl.ds(sid * TBL_S, TBL_S)])
        plsc.subcore_barrier()

    with jax.named_scope("stage_tile"):
        pltpu.sync_copy(tbl_sh, table_v)

    for c in range(NCHUNK):
        with jax.named_scope("chunk"):
            cps[c].wait()
            if c + 2 < NCHUNK:
                cps[c + 2] = start(c + 2)
            idx_v = idx_bufs[c % 2]

            @plsc.parallel_loop(0, GROUPS_PER_CHUNK, 1)
            def group(g):
                rows = g * L + lanes
                acc = [jnp.ones((L,), jnp.float32) for _ in range(NACC)]
                for j in range(H):
                    iv = plsc.load_gather(
                        idx_v, [rows, jnp.full((L,), j, jnp.int32)])
                    v = plsc.load_gather(table_v, [iv])
                    acc[j % NACC] = acc[j % NACC] * v
                p = (acc[0] * acc[1]) * (acc[2] * acc[3])
                no = 1.0 - 1.0 / p
                no = jnp.minimum(jnp.maximum(no, 0.0001), 0.99999)
                out_v[pl.ds(c * CHUNK + g * L, L)] = no

    with jax.named_scope("store_out"):
        pltpu.sync_copy(out_v, out_hbm.at[pl.ds(base, ROWS_PER_W)])


@jax.jit
def kernel(rules, relation, table):
    del relation
    tbl = 1.0 + jnp.exp(table[:, 0].at[PAD_TOK].set(-jnp.inf))
    tbl = jnp.concatenate([tbl, jnp.ones((TBL_PAD - (LEN_RULES + 1),), jnp.float32)])
    run = pl.kernel(
        _body,
        out_type=jax.ShapeDtypeStruct((B,), jnp.float32),
        mesh=plsc.VectorSubcoreMesh(
            core_axis_name="c", subcore_axis_name="s",
            num_cores=NC, num_subcores=NS,
        ),
        compiler_params=pltpu.CompilerParams(
            needs_layout_passes=False, use_tc_tiling_on_sc=True),
        scratch_types=[
            pltpu.VMEM((TBL_PAD,), jnp.float32),
            pltpu.VMEM((CHUNK, H), jnp.int32),
            pltpu.VMEM((CHUNK, H), jnp.int32),
            pltpu.VMEM((ROWS_PER_W,), jnp.float32),
            pltpu.VMEM_SHARED((TBL_PAD,), jnp.float32),
            pltpu.SemaphoreType.DMA,
            pltpu.SemaphoreType.DMA,
            pltpu.SemaphoreType.DMA,
        ],
    )
    return run(rules, tbl).reshape(B, 1)
